# Initial kernel scaffold; baseline (speedup 1.0000x reference)
import functools
import jax
import jax.numpy as jnp
from jax import lax
from jax.experimental import pallas as pl
from jax.experimental.pallas import tpu as pltpu
from jax.experimental.pallas import tpu_sc as plsc

N = 100000
E = 1600000
C = 5
RADIUS = 0.05
ACC_SCALE = 0.4
MAX_VEL = 0.1

NC = 2
NS = 16
NW = NC * NS
EPW = E // NW
ROWS_PER_TILE = N // NS
CHUNK = 2000
NCHUNK = EPW // CHUNK
SUB = 80
NSUB = CHUNK // SUB
GROUPS = CHUNK // 16


def _y_body(x_ref, w_ref, y_ref):
    y_ref[...] = jnp.dot(x_ref[...], w_ref[...],
                         preferred_element_type=jnp.float32)


def _compute_y(x, wcat):
    blk = 2500
    grid = N // blk
    return pl.pallas_call(
        _y_body,
        grid=(grid,),
        in_specs=[
            pl.BlockSpec((blk, C), lambda i: (i, 0)),
            pl.BlockSpec((C, 16), lambda i: (0, 0)),
        ],
        out_specs=pl.BlockSpec((blk, 16), lambda i: (i, 0)),
        out_shape=jax.ShapeDtypeStruct((N, 16), jnp.float32),
    )(x, wcat)


def _edge_body(y_hbm, src_hbm, dst_hbm, d_hbm, c_hbm, w1_hbm, z_hbm,
               part_hbm, acc_sp, y_v, msg_v, cnt_v, si_v, di_v, d_v, c_v,
               w_v, sem):
    cid = lax.axis_index("c")
    sid = lax.axis_index("s")
    wid = cid * NS + sid

    pltpu.sync_copy(z_hbm, acc_sp.at[pl.ds(sid * ROWS_PER_TILE, ROWS_PER_TILE)])
    pltpu.sync_copy(z_hbm.at[pl.ds(0, CHUNK)], msg_v)
    pltpu.sync_copy(z_hbm.at[pl.ds(0, CHUNK)], cnt_v)
    pltpu.sync_copy(w1_hbm, w_v)
    w00 = w_v[0]
    w01 = w_v[1]
    b10 = w_v[2]
    w10 = w_v[3]
    w11 = w_v[4]
    b11 = w_v[5]
    iota16 = lax.iota(jnp.int32, 16)
    plsc.subcore_barrier()

    def chunk_body(k, carry):
        rowbase = wid * (EPW // SUB) + k * NSUB
        ebase = wid * EPW + k * CHUNK
        pltpu.sync_copy(src_hbm.at[pl.ds(rowbase, NSUB)], si_v)
        pltpu.sync_copy(dst_hbm.at[pl.ds(rowbase, NSUB)], di_v)
        pltpu.sync_copy(d_hbm.at[pl.ds(ebase, CHUNK)], d_v)
        pltpu.sync_copy(c_hbm.at[pl.ds(ebase, CHUNK)], c_v)
        descs = [
            pltpu.async_copy(y_hbm.at[si_v.at[j]],
                             y_v.at[pl.ds(j * SUB, SUB)], sem)
            for j in range(NSUB)
        ]
        for dsc in descs:
            dsc.wait()

        def grp(g, c2):
            dd = d_v[pl.ds(g * 16, 16)]
            cc = c_v[pl.ds(g * 16, 16)]
            rows = g * 16 + iota16
            h0 = jnp.maximum(w00 * dd + w01 * cc + b10, 0.0)
            h1 = jnp.maximum(w10 * dd + w11 * cc + b11, 0.0)
            for o in range(C):
                co = jnp.full((16,), o, jnp.int32)
                a0 = plsc.load_gather(y_v, [rows, co])
                a1 = plsc.load_gather(y_v, [rows, co + 5])
                a2 = plsc.load_gather(y_v, [rows, co + 10])
                m = a0 + h0 * a1 + h1 * a2
                plsc.store_scatter(msg_v, [rows, co], m)
            cell = jnp.where(cc == 1.0, 1.0, 0.0)
            close = jnp.where(dd < RADIUS, 1.0, 0.0)
            plsc.store_scatter(cnt_v, [rows, jnp.full((16,), 5, jnp.int32)],
                               cell)
            plsc.store_scatter(cnt_v, [rows, jnp.full((16,), 6, jnp.int32)],
                               close)
            return c2

        lax.fori_loop(0, GROUPS, grp, 0)
        for j in range(NSUB):
            pltpu.sync_copy(msg_v.at[pl.ds(j * SUB, SUB)],
                            acc_sp.at[di_v.at[j]], add=True)
            pltpu.sync_copy(cnt_v.at[pl.ds(j * SUB, SUB)],
                            acc_sp.at[si_v.at[j]], add=True)
        return carry

    lax.fori_loop(0, NCHUNK, chunk_body, 0)
    plsc.subcore_barrier()
    pltpu.sync_copy(
        acc_sp.at[pl.ds(sid * ROWS_PER_TILE, ROWS_PER_TILE)],
        part_hbm.at[cid].at[pl.ds(sid * ROWS_PER_TILE, ROWS_PER_TILE)])


def _edge_phase(y, srcm, dstm, d, c, w1p, zeros):
    mesh = plsc.VectorSubcoreMesh(core_axis_name="c", subcore_axis_name="s")
    kb = pl.kernel(
        _edge_body,
        out_type=jax.ShapeDtypeStruct((NC, N, 8), jnp.float32),
        mesh=mesh,
        scratch_types=[
            pltpu.VMEM_SHARED((N, 8), jnp.float32),
            pltpu.VMEM((CHUNK, 16), jnp.float32),
            pltpu.VMEM((CHUNK, 8), jnp.float32),
            pltpu.VMEM((CHUNK, 8), jnp.float32),
            pltpu.VMEM((NSUB, SUB), jnp.int32),
            pltpu.VMEM((NSUB, SUB), jnp.int32),
            pltpu.VMEM((CHUNK,), jnp.float32),
            pltpu.VMEM((CHUNK,), jnp.float32),
            pltpu.VMEM((16,), jnp.float32),
            pltpu.SemaphoreType.DMA,
        ],
    )
    return kb(y, srcm, dstm, d, c, w1p, zeros)


def _node_body(x_ref, p0_ref, p1_ref, root_ref, bias_ref, m1wt_ref, m1b_ref,
               m3wt_ref, m3b_ref, newx_ref, border_ref, food_ref, dead_ref):
    i = pl.program_id(0)
    xb = x_ref[...]
    agg = p0_ref[...] + p1_ref[...]
    aggr = agg[:, :C]
    ccnt = agg[:, 5]
    clcnt = agg[:, 6]
    conv = (jnp.dot(xb, root_ref[...], preferred_element_type=jnp.float32)
            + aggr + bias_ref[...])
    hh = jnp.maximum(
        jnp.dot(conv, m1wt_ref[...], preferred_element_type=jnp.float32)
        + m1b_ref[...], 0.0)
    out2 = (jnp.dot(hh, m3wt_ref[...], preferred_element_type=jnp.float32)
            + m3b_ref[...])
    x4 = xb[:, 4]
    fmask = (x4 == 1.0).astype(jnp.float32)
    acc = out2 * ACC_SCALE * fmask[:, None]
    vel = jnp.clip(xb[:, 2:4] + acc, -MAX_VEL, MAX_VEL)
    pos = xb[:, 0:2] + vel
    newx_ref[...] = jnp.concatenate([pos, vel, x4[:, None]], axis=1)
    eps = 1e-06
    ax = jnp.abs(pos[:, 0])
    ay = jnp.abs(pos[:, 1])
    bc = (jnp.sum(jnp.where(ax > 1.0, jnp.log(ax + eps), 0.0))
          + jnp.sum(jnp.where(ay > 1.0, jnp.log(ay + eps), 0.0)))
    dead = jnp.sum(jnp.where((x4 == 1.0) & (ccnt < 1.0), 1.0, 0.0))
    food = jnp.sum(jnp.where((x4 == 0.0) & (clcnt >= 3.0), 1.0, 0.0))

    @pl.when(i == 0)
    def _():
        border_ref[0, 0] = 0.0
        food_ref[0, 0] = 0.0
        dead_ref[0, 0] = 0.0

    border_ref[0, 0] += bc
    food_ref[0, 0] += food
    dead_ref[0, 0] += dead


def _node_phase(x, p0, p1, root, bias, m1wt, m1b, m3wt, m3b):
    blk = 2500
    grid = N // blk
    full = lambda i: (0, 0)
    return pl.pallas_call(
        _node_body,
        grid=(grid,),
        in_specs=[
            pl.BlockSpec((blk, C), lambda i: (i, 0)),
            pl.BlockSpec((blk, 8), lambda i: (i, 0)),
            pl.BlockSpec((blk, 8), lambda i: (i, 0)),
            pl.BlockSpec((C, C), full),
            pl.BlockSpec((1, C), full),
            pl.BlockSpec((C, C), full),
            pl.BlockSpec((1, C), full),
            pl.BlockSpec((C, 2), full),
            pl.BlockSpec((1, 2), full),
        ],
        out_specs=[
            pl.BlockSpec((blk, C), lambda i: (i, 0)),
            pl.BlockSpec((1, 1), full),
            pl.BlockSpec((1, 1), full),
            pl.BlockSpec((1, 1), full),
        ],
        out_shape=[
            jax.ShapeDtypeStruct((N, C), jnp.float32),
            jax.ShapeDtypeStruct((1, 1), jnp.float32),
            jax.ShapeDtypeStruct((1, 1), jnp.float32),
            jax.ShapeDtypeStruct((1, 1), jnp.float32),
        ],
    )(x, p0, p1, root, bias, m1wt, m1b, m3wt, m3b)


@jax.jit
def kernel(x, edge_index, edge_attr, W1, b1, W2, b2, root, bias,
           m1w, m1b, m3w, m3b):
    src = edge_index[0].astype(jnp.int32).reshape(E // SUB, SUB)
    dst = edge_index[1].astype(jnp.int32).reshape(E // SUB, SUB)
    d = edge_attr[:, 0]
    c = edge_attr[:, 1]

    b2r = b2.reshape(C, C)
    w2r0 = W2[:, 0].reshape(C, C)
    w2r1 = W2[:, 1].reshape(C, C)
    wcat = jnp.concatenate(
        [b2r, w2r0, w2r1, jnp.zeros((C, 1), jnp.float32)], axis=1)

    w1p = jnp.zeros((16,), jnp.float32)
    w1p = w1p.at[0].set(W1[0, 0]).at[1].set(W1[0, 1]).at[2].set(b1[0])
    w1p = w1p.at[3].set(W1[1, 0]).at[4].set(W1[1, 1]).at[5].set(b1[1])

    zeros = jnp.zeros((ROWS_PER_TILE, 8), jnp.float32)

    y = _compute_y(x, wcat)
    part = _edge_phase(y, src, dst, d, c, w1p, zeros)

    newx, border, food, dead = _node_phase(
        x, part[0], part[1], root, bias.reshape(1, C),
        m1w.T, m1b.reshape(1, C), m3w.T, m3b.reshape(1, 2))
    return newx, border[0, 0], food[0, 0], dead[0, 0]


# speedup vs baseline: 13.8938x; 13.8938x over previous
import functools
import jax
import jax.numpy as jnp
from jax import lax
from jax.experimental import pallas as pl
from jax.experimental.pallas import tpu as pltpu
from jax.experimental.pallas import tpu_sc as plsc

N = 100000
E = 1600000
C = 5
RADIUS = 0.05
ACC_SCALE = 0.4
MAX_VEL = 0.1

NC = 2
NS = 16
NW = NC * NS
EPW = E // NW
ROWS_PER_TILE = 6272
N_PAD = NS * ROWS_PER_TILE
CHUNK = 2000
NCHUNK = EPW // CHUNK
SUB = 125
NSUB = CHUNK // SUB
GROUPS = CHUNK // 16


def _y_body(x_ref, w_ref, y_ref):
    y_ref[...] = jnp.dot(x_ref[...], w_ref[...],
                         preferred_element_type=jnp.float32)


def _compute_y(x, wcat):
    blk = 4000
    grid = N // blk
    return pl.pallas_call(
        _y_body,
        grid=(grid,),
        in_specs=[
            pl.BlockSpec((blk, C), lambda i: (i, 0)),
            pl.BlockSpec((C, 16), lambda i: (0, 0)),
        ],
        out_specs=pl.BlockSpec((blk, 16), lambda i: (i, 0)),
        out_shape=jax.ShapeDtypeStruct((N, 16), jnp.float32),
    )(x, wcat)


def _edge_body(y_hbm, src_hbm, dst_hbm, d_hbm, c_hbm, w1_hbm, z_hbm,
               part_hbm, acc_sp, y_v, msg_v, cnt_v, si_v, di_v, d_v, c_v,
               w_v, sem):
    cid = lax.axis_index("c")
    sid = lax.axis_index("s")
    wid = cid * NS + sid

    pltpu.sync_copy(z_hbm, acc_sp.at[pl.ds(sid * ROWS_PER_TILE, ROWS_PER_TILE)])
    pltpu.sync_copy(z_hbm.at[pl.ds(0, CHUNK)], msg_v)
    pltpu.sync_copy(z_hbm.at[pl.ds(0, CHUNK)], cnt_v)
    pltpu.sync_copy(w1_hbm, w_v)
    wvec = w_v[...]
    w00 = wvec[0]
    w01 = wvec[1]
    b10 = wvec[2]
    w10 = wvec[3]
    w11 = wvec[4]
    b11 = wvec[5]
    iota16 = lax.iota(jnp.int32, 16)
    plsc.subcore_barrier()

    def chunk_body(k, carry):
        rowbase = wid * (EPW // SUB) + k * NSUB
        ebase = wid * EPW + k * CHUNK
        pltpu.sync_copy(src_hbm.at[pl.ds(rowbase, NSUB)], si_v)
        pltpu.sync_copy(dst_hbm.at[pl.ds(rowbase, NSUB)], di_v)
        pltpu.sync_copy(d_hbm.at[pl.ds(ebase, CHUNK)], d_v)
        pltpu.sync_copy(c_hbm.at[pl.ds(ebase, CHUNK)], c_v)
        descs = [
            pltpu.async_copy(y_hbm.at[si_v.at[j]],
                             y_v.at[pl.ds(j * SUB, SUB)], sem)
            for j in range(NSUB)
        ]
        for dsc in descs:
            dsc.wait()

        def grp(g, c2):
            dd = d_v[pl.ds(g * 16, 16)]
            cc = c_v[pl.ds(g * 16, 16)]
            rows = g * 16 + iota16
            h0 = jnp.maximum(w00 * dd + w01 * cc + b10, 0.0)
            h1 = jnp.maximum(w10 * dd + w11 * cc + b11, 0.0)
            for o in range(C):
                co = jnp.full((16,), o, jnp.int32)
                a0 = plsc.load_gather(y_v, [rows, co])
                a1 = plsc.load_gather(y_v, [rows, co + 5])
                a2 = plsc.load_gather(y_v, [rows, co + 10])
                m = a0 + h0 * a1 + h1 * a2
                plsc.store_scatter(msg_v, [rows, co], m)
            cell = jnp.where(cc == 1.0, 1.0, 0.0)
            close = jnp.where(dd < RADIUS, 1.0, 0.0)
            plsc.store_scatter(cnt_v, [rows, jnp.full((16,), 5, jnp.int32)],
                               cell)
            plsc.store_scatter(cnt_v, [rows, jnp.full((16,), 6, jnp.int32)],
                               close)
            return c2

        lax.fori_loop(0, GROUPS, grp, 0)
        for j in range(NSUB):
            pltpu.sync_copy(msg_v.at[pl.ds(j * SUB, SUB)],
                            acc_sp.at[di_v.at[j]], add=True)
            pltpu.sync_copy(cnt_v.at[pl.ds(j * SUB, SUB)],
                            acc_sp.at[si_v.at[j]], add=True)
        return carry

    lax.fori_loop(0, NCHUNK, chunk_body, 0)
    plsc.subcore_barrier()
    pltpu.sync_copy(
        acc_sp.at[pl.ds(sid * ROWS_PER_TILE, ROWS_PER_TILE)],
        part_hbm.at[cid].at[pl.ds(sid * ROWS_PER_TILE, ROWS_PER_TILE)])


def _edge_phase(y, srcm, dstm, d, c, w1p, zeros):
    mesh = plsc.VectorSubcoreMesh(core_axis_name="c", subcore_axis_name="s")
    kb = pl.kernel(
        _edge_body,
        out_type=jax.ShapeDtypeStruct((NC, N_PAD, 8), jnp.float32),
        mesh=mesh,
        scratch_types=[
            pltpu.VMEM_SHARED((N_PAD, 8), jnp.float32),
            pltpu.VMEM((CHUNK, 16), jnp.float32),
            pltpu.VMEM((CHUNK, 8), jnp.float32),
            pltpu.VMEM((CHUNK, 8), jnp.float32),
            pltpu.VMEM((NSUB, SUB), jnp.int32),
            pltpu.VMEM((NSUB, SUB), jnp.int32),
            pltpu.VMEM((CHUNK,), jnp.float32),
            pltpu.VMEM((CHUNK,), jnp.float32),
            pltpu.VMEM((16,), jnp.float32),
            pltpu.SemaphoreType.DMA,
        ],
        compiler_params=pltpu.CompilerParams(needs_layout_passes=False,
                                             use_tc_tiling_on_sc=False),
    )
    return kb(y, srcm, dstm, d, c, w1p, zeros)


def _node_body(x_ref, p0_ref, p1_ref, root_ref, bias_ref, m1wt_ref, m1b_ref,
               m3wt_ref, m3b_ref, newx_ref, border_ref, food_ref, dead_ref):
    i = pl.program_id(0)
    xb = x_ref[...]
    agg = p0_ref[...] + p1_ref[...]
    aggr = agg[:, :C]
    ccnt = agg[:, 5]
    clcnt = agg[:, 6]
    conv = (jnp.dot(xb, root_ref[...], preferred_element_type=jnp.float32)
            + aggr + bias_ref[...])
    hh = jnp.maximum(
        jnp.dot(conv, m1wt_ref[...], preferred_element_type=jnp.float32)
        + m1b_ref[...], 0.0)
    out2 = (jnp.dot(hh, m3wt_ref[...], preferred_element_type=jnp.float32)
            + m3b_ref[...])
    x4 = xb[:, 4]
    fmask = (x4 == 1.0).astype(jnp.float32)
    acc = out2 * ACC_SCALE * fmask[:, None]
    vel = jnp.clip(xb[:, 2:4] + acc, -MAX_VEL, MAX_VEL)
    pos = xb[:, 0:2] + vel
    newx_ref[...] = jnp.concatenate([pos, vel, x4[:, None]], axis=1)
    eps = 1e-06
    ax = jnp.abs(pos[:, 0])
    ay = jnp.abs(pos[:, 1])
    bc = (jnp.sum(jnp.where(ax > 1.0, jnp.log(ax + eps), 0.0))
          + jnp.sum(jnp.where(ay > 1.0, jnp.log(ay + eps), 0.0)))
    dead = jnp.sum(jnp.where((x4 == 1.0) & (ccnt < 1.0), 1.0, 0.0))
    food = jnp.sum(jnp.where((x4 == 0.0) & (clcnt >= 3.0), 1.0, 0.0))

    zero11 = jnp.zeros((1, 1), jnp.float32)

    @pl.when(i == 0)
    def _():
        border_ref[...] = zero11
        food_ref[...] = zero11
        dead_ref[...] = zero11

    border_ref[...] += bc.reshape(1, 1)
    food_ref[...] += food.reshape(1, 1)
    dead_ref[...] += dead.reshape(1, 1)


def _node_phase(x, p0, p1, root, bias, m1wt, m1b, m3wt, m3b):
    blk = 4000
    grid = N // blk
    full = lambda i: (0, 0)
    return pl.pallas_call(
        _node_body,
        grid=(grid,),
        in_specs=[
            pl.BlockSpec((blk, C), lambda i: (i, 0)),
            pl.BlockSpec((blk, 8), lambda i: (i, 0)),
            pl.BlockSpec((blk, 8), lambda i: (i, 0)),
            pl.BlockSpec((C, C), full),
            pl.BlockSpec((1, C), full),
            pl.BlockSpec((C, C), full),
            pl.BlockSpec((1, C), full),
            pl.BlockSpec((C, 2), full),
            pl.BlockSpec((1, 2), full),
        ],
        out_specs=[
            pl.BlockSpec((blk, C), lambda i: (i, 0)),
            pl.BlockSpec((1, 1), full),
            pl.BlockSpec((1, 1), full),
            pl.BlockSpec((1, 1), full),
        ],
        out_shape=[
            jax.ShapeDtypeStruct((N, C), jnp.float32),
            jax.ShapeDtypeStruct((1, 1), jnp.float32),
            jax.ShapeDtypeStruct((1, 1), jnp.float32),
            jax.ShapeDtypeStruct((1, 1), jnp.float32),
        ],
    )(x, p0, p1, root, bias, m1wt, m1b, m3wt, m3b)


@jax.jit
def kernel(x, edge_index, edge_attr, W1, b1, W2, b2, root, bias,
           m1w, m1b, m3w, m3b):
    src = edge_index[0].astype(jnp.int32).reshape(E // SUB, SUB)
    dst = edge_index[1].astype(jnp.int32).reshape(E // SUB, SUB)
    d = edge_attr[:, 0]
    c = edge_attr[:, 1]

    b2r = b2.reshape(C, C)
    w2r0 = W2[:, 0].reshape(C, C)
    w2r1 = W2[:, 1].reshape(C, C)
    wcat = jnp.concatenate(
        [b2r, w2r0, w2r1, jnp.zeros((C, 1), jnp.float32)], axis=1)

    w1p = jnp.zeros((16,), jnp.float32)
    w1p = w1p.at[0].set(W1[0, 0]).at[1].set(W1[0, 1]).at[2].set(b1[0])
    w1p = w1p.at[3].set(W1[1, 0]).at[4].set(W1[1, 1]).at[5].set(b1[1])

    zeros = jnp.zeros((ROWS_PER_TILE, 8), jnp.float32)

    y = _compute_y(x, wcat)
    part = _edge_phase(y, src, dst, d, c, w1p, zeros)

    newx, border, food, dead = _node_phase(
        x, part[0, :N], part[1, :N], root, bias.reshape(1, C),
        m1w.T, m1b.reshape(1, C), m3w.T, m3b.reshape(1, 2))
    return newx, border[0, 0], food[0, 0], dead[0, 0]


# speedup vs baseline: 14.8713x; 1.0704x over previous
import functools
import jax
import jax.numpy as jnp
from jax import lax
from jax.experimental import pallas as pl
from jax.experimental.pallas import tpu as pltpu
from jax.experimental.pallas import tpu_sc as plsc

N = 100000
E = 1600000
C = 5
RADIUS = 0.05
ACC_SCALE = 0.4
MAX_VEL = 0.1

NC = 2
NS = 16
NW = NC * NS
EPW = E // NW
ROWS_PER_TILE = 6272
N_PAD = NS * ROWS_PER_TILE
CHUNK = 2000
NCHUNK = EPW // CHUNK
SUB = 125
NSUB = CHUNK // SUB
GROUPS = CHUNK // 16


def _y_body(x_ref, w_ref, y_ref):
    y_ref[...] = jnp.dot(x_ref[...], w_ref[...],
                         preferred_element_type=jnp.float32)


def _compute_y(x, wcat):
    blk = 4000
    grid = N // blk
    return pl.pallas_call(
        _y_body,
        grid=(grid,),
        in_specs=[
            pl.BlockSpec((blk, C), lambda i: (i, 0)),
            pl.BlockSpec((C, 16), lambda i: (0, 0)),
        ],
        out_specs=pl.BlockSpec((blk, 16), lambda i: (i, 0)),
        out_shape=jax.ShapeDtypeStruct((N, 16), jnp.float32),
    )(x, wcat)


def _edge_body(y_hbm, src_hbm, dst_hbm, d_hbm, c_hbm, w1_hbm, z_hbm,
               part_hbm, acc_sp, y_v, msg_v, cnt_v, si_v, di_v, d_v, c_v,
               w_v, sem):
    cid = lax.axis_index("c")
    sid = lax.axis_index("s")
    wid = cid * NS + sid

    pltpu.sync_copy(z_hbm, acc_sp.at[pl.ds(sid * ROWS_PER_TILE, ROWS_PER_TILE)])
    pltpu.sync_copy(z_hbm.at[pl.ds(0, CHUNK)], msg_v)
    pltpu.sync_copy(z_hbm.at[pl.ds(0, CHUNK)], cnt_v)
    pltpu.sync_copy(w1_hbm, w_v)
    wvec = w_v[...]
    w00 = wvec[0]
    w01 = wvec[1]
    b10 = wvec[2]
    w10 = wvec[3]
    w11 = wvec[4]
    b11 = wvec[5]
    iota16 = lax.iota(jnp.int32, 16)
    plsc.subcore_barrier()

    def chunk_body(k, carry):
        rowbase = wid * (EPW // SUB) + k * NSUB
        ebase = wid * EPW + k * CHUNK
        loads = [
            pltpu.async_copy(src_hbm.at[pl.ds(rowbase, NSUB)], si_v, sem),
            pltpu.async_copy(dst_hbm.at[pl.ds(rowbase, NSUB)], di_v, sem),
            pltpu.async_copy(d_hbm.at[pl.ds(ebase, CHUNK)], d_v, sem),
            pltpu.async_copy(c_hbm.at[pl.ds(ebase, CHUNK)], c_v, sem),
        ]
        for dsc in loads:
            dsc.wait()
        descs = [
            pltpu.async_copy(y_hbm.at[si_v.at[j]],
                             y_v.at[pl.ds(j * SUB, SUB)], sem)
            for j in range(NSUB)
        ]
        for dsc in descs:
            dsc.wait()

        def grp(g, c2):
            dd = d_v[pl.ds(g * 16, 16)]
            cc = c_v[pl.ds(g * 16, 16)]
            rows = g * 16 + iota16
            h0 = jnp.maximum(w00 * dd + w01 * cc + b10, 0.0)
            h1 = jnp.maximum(w10 * dd + w11 * cc + b11, 0.0)
            for o in range(C):
                co = jnp.full((16,), o, jnp.int32)
                a0 = plsc.load_gather(y_v, [rows, co])
                a1 = plsc.load_gather(y_v, [rows, co + 5])
                a2 = plsc.load_gather(y_v, [rows, co + 10])
                m = a0 + h0 * a1 + h1 * a2
                plsc.store_scatter(msg_v, [rows, co], m)
            cell = jnp.where(cc == 1.0, 1.0, 0.0)
            close = jnp.where(dd < RADIUS, 1.0, 0.0)
            plsc.store_scatter(cnt_v, [rows, jnp.full((16,), 5, jnp.int32)],
                               cell)
            plsc.store_scatter(cnt_v, [rows, jnp.full((16,), 6, jnp.int32)],
                               close)
            return c2

        lax.fori_loop(0, GROUPS, grp, 0)
        scats = []
        for j in range(NSUB):
            scats.append(pltpu.async_copy(
                msg_v.at[pl.ds(j * SUB, SUB)], acc_sp.at[di_v.at[j]],
                sem, add=True))
            scats.append(pltpu.async_copy(
                cnt_v.at[pl.ds(j * SUB, SUB)], acc_sp.at[si_v.at[j]],
                sem, add=True))
        for dsc in scats:
            dsc.wait()
        return carry

    lax.fori_loop(0, NCHUNK, chunk_body, 0)
    plsc.subcore_barrier()
    pltpu.sync_copy(
        acc_sp.at[pl.ds(sid * ROWS_PER_TILE, ROWS_PER_TILE)],
        part_hbm.at[cid].at[pl.ds(sid * ROWS_PER_TILE, ROWS_PER_TILE)])


def _edge_phase(y, srcm, dstm, d, c, w1p, zeros):
    mesh = plsc.VectorSubcoreMesh(core_axis_name="c", subcore_axis_name="s")
    kb = pl.kernel(
        _edge_body,
        out_type=jax.ShapeDtypeStruct((NC, N_PAD, 8), jnp.float32),
        mesh=mesh,
        scratch_types=[
            pltpu.VMEM_SHARED((N_PAD, 8), jnp.float32),
            pltpu.VMEM((CHUNK, 16), jnp.float32),
            pltpu.VMEM((CHUNK, 8), jnp.float32),
            pltpu.VMEM((CHUNK, 8), jnp.float32),
            pltpu.VMEM((NSUB, SUB), jnp.int32),
            pltpu.VMEM((NSUB, SUB), jnp.int32),
            pltpu.VMEM((CHUNK,), jnp.float32),
            pltpu.VMEM((CHUNK,), jnp.float32),
            pltpu.VMEM((16,), jnp.float32),
            pltpu.SemaphoreType.DMA,
        ],
        compiler_params=pltpu.CompilerParams(needs_layout_passes=False,
                                             use_tc_tiling_on_sc=False),
    )
    return kb(y, srcm, dstm, d, c, w1p, zeros)


def _node_body(x_ref, p0_ref, p1_ref, rbd_ref, biast_ref, m1bd_ref, m1bt_ref,
               m3bd_ref, m3bt_ref, sbd_ref, pbd_ref,
               newx_ref, border_ref, food_ref, dead_ref):
    i = pl.program_id(0)
    xp = x_ref[...]
    ap = p0_ref[...] + p1_ref[...]
    lane = lax.broadcasted_iota(jnp.int32, xp.shape, 1) % 8
    conv = (jnp.dot(xp, rbd_ref[...], preferred_element_type=jnp.float32)
            + jnp.where(lane < C, ap, 0.0) + biast_ref[...])
    hh = jnp.maximum(
        jnp.dot(conv, m1bd_ref[...], preferred_element_type=jnp.float32)
        + m1bt_ref[...], 0.0)
    out2 = (jnp.dot(hh, m3bd_ref[...], preferred_element_type=jnp.float32)
            + m3bt_ref[...])
    x4b = jnp.dot(xp, sbd_ref[...], preferred_element_type=jnp.float32)
    fm = (x4b == 1.0)
    accp = out2 * ACC_SCALE * jnp.where(fm, 1.0, 0.0)
    velp = jnp.where((lane == 2) | (lane == 3),
                     jnp.clip(xp + accp, -MAX_VEL, MAX_VEL), 0.0)
    posp = jnp.where(lane < 2,
                     xp + jnp.dot(velp, pbd_ref[...],
                                  preferred_element_type=jnp.float32), 0.0)
    newx_ref[...] = posp + velp + jnp.where(lane == 4, xp, 0.0)
    eps = 1e-06
    axp = jnp.abs(posp)
    bc = jnp.sum(jnp.where((axp > 1.0) & (lane < 2), jnp.log(axp + eps), 0.0))
    dead = jnp.sum(jnp.where(fm & (ap < 1.0) & (lane == 5), 1.0, 0.0))
    food = jnp.sum(jnp.where((x4b == 0.0) & (ap >= 3.0) & (lane == 6),
                             1.0, 0.0))

    zero11 = jnp.zeros((1, 1), jnp.float32)

    @pl.when(i == 0)
    def _():
        border_ref[...] = zero11
        food_ref[...] = zero11
        dead_ref[...] = zero11

    border_ref[...] += bc.reshape(1, 1)
    food_ref[...] += food.reshape(1, 1)
    dead_ref[...] += dead.reshape(1, 1)


PACK_ROWS = N_PAD // 16


def _node_phase(xp, p0p, p1p, rbd, biast, m1bd, m1bt, m3bd, m3bt, sbd, pbd):
    blk = 1568
    grid = PACK_ROWS // blk
    full = lambda i: (0, 0)
    wspec = pl.BlockSpec((128, 128), full)
    bspec = pl.BlockSpec((1, 128), full)
    return pl.pallas_call(
        _node_body,
        grid=(grid,),
        in_specs=[
            pl.BlockSpec((blk, 128), lambda i: (i, 0)),
            pl.BlockSpec((blk, 128), lambda i: (i, 0)),
            pl.BlockSpec((blk, 128), lambda i: (i, 0)),
            wspec, bspec, wspec, bspec, wspec, bspec, wspec, wspec,
        ],
        out_specs=[
            pl.BlockSpec((blk, 128), lambda i: (i, 0)),
            pl.BlockSpec((1, 1), full),
            pl.BlockSpec((1, 1), full),
            pl.BlockSpec((1, 1), full),
        ],
        out_shape=[
            jax.ShapeDtypeStruct((PACK_ROWS, 128), jnp.float32),
            jax.ShapeDtypeStruct((1, 1), jnp.float32),
            jax.ShapeDtypeStruct((1, 1), jnp.float32),
            jax.ShapeDtypeStruct((1, 1), jnp.float32),
        ],
    )(xp, p0p, p1p, rbd, biast, m1bd, m1bt, m3bd, m3bt, sbd, pbd)


@jax.jit
def kernel(x, edge_index, edge_attr, W1, b1, W2, b2, root, bias,
           m1w, m1b, m3w, m3b):
    src = edge_index[0].astype(jnp.int32).reshape(E // SUB, SUB)
    dst = edge_index[1].astype(jnp.int32).reshape(E // SUB, SUB)
    d = edge_attr[:, 0]
    c = edge_attr[:, 1]

    b2r = b2.reshape(C, C)
    w2r0 = W2[:, 0].reshape(C, C)
    w2r1 = W2[:, 1].reshape(C, C)
    wcat = jnp.concatenate(
        [b2r, w2r0, w2r1, jnp.zeros((C, 1), jnp.float32)], axis=1)

    w1p = jnp.zeros((16,), jnp.float32)
    w1p = w1p.at[0].set(W1[0, 0]).at[1].set(W1[0, 1]).at[2].set(b1[0])
    w1p = w1p.at[3].set(W1[1, 0]).at[4].set(W1[1, 1]).at[5].set(b1[1])

    zeros = jnp.zeros((ROWS_PER_TILE, 8), jnp.float32)

    y = _compute_y(x, wcat)
    part = _edge_phase(y, src, dst, d, c, w1p, zeros)

    eye16 = jnp.eye(16, dtype=jnp.float32)
    r8 = jnp.zeros((8, 8), jnp.float32).at[:C, :C].set(root)
    m18 = jnp.zeros((8, 8), jnp.float32).at[:C, :C].set(m1w.T)
    m38 = jnp.zeros((8, 8), jnp.float32).at[:C, 2:4].set(m3w.T)
    s8 = jnp.zeros((8, 8), jnp.float32).at[4, :].set(1.0)
    p8 = jnp.zeros((8, 8), jnp.float32).at[2, 0].set(1.0).at[3, 1].set(1.0)
    rbd = jnp.kron(eye16, r8)
    m1bd = jnp.kron(eye16, m18)
    m3bd = jnp.kron(eye16, m38)
    sbd = jnp.kron(eye16, s8)
    pbd = jnp.kron(eye16, p8)
    pad8 = lambda v: jnp.concatenate([v, jnp.zeros((3,), jnp.float32)])
    biast = jnp.tile(pad8(bias), 16).reshape(1, 128)
    m1bt = jnp.tile(pad8(m1b), 16).reshape(1, 128)
    m3bt = jnp.tile(
        jnp.zeros((8,), jnp.float32).at[2].set(m3b[0]).at[3].set(m3b[1]),
        16).reshape(1, 128)

    xpad = jnp.zeros((N_PAD, 8), jnp.float32).at[:N, :C].set(x)
    xp = xpad.reshape(PACK_ROWS, 128)
    p0p = part[0].reshape(PACK_ROWS, 128)
    p1p = part[1].reshape(PACK_ROWS, 128)

    newxp, border, food, dead = _node_phase(
        xp, p0p, p1p, rbd, biast, m1bd, m1bt, m3bd, m3bt, sbd, pbd)
    newx = newxp.reshape(N_PAD, 8)[:N, :C]
    return newx, border[0, 0], food[0, 0], dead[0, 0]
